# baseline (speedup 1.0000x reference)
import functools

import jax
import jax.numpy as jnp
from jax import lax
from jax.experimental import pallas as pl
from jax.experimental.pallas import tpu as pltpu

HIDDEN_DIM = 64
BATCH = 16384


_NQ = 8


def _tc_body(idx_ref, table_ref, out_ref, *sems):
    def body(g, carry):
        for q in range(_NQ):
            i = g * _NQ + q
            j = idx_ref[i]
            pltpu.make_async_copy(
                table_ref.at[j], out_ref.at[i], sems[q]
            ).start()
        return carry

    lax.fori_loop(0, BATCH // _NQ, body, 0)
    for q in range(_NQ):
        pltpu.make_async_copy(
            table_ref.at[pl.ds(0, BATCH // _NQ)],
            out_ref.at[pl.ds(0, BATCH // _NQ)],
            sems[q],
        ).wait()


_tc_gather = pl.pallas_call(
    _tc_body,
    out_shape=jax.ShapeDtypeStruct((BATCH, HIDDEN_DIM), jnp.float32),
    in_specs=[
        pl.BlockSpec(memory_space=pltpu.SMEM),
        pl.BlockSpec(memory_space=pl.ANY),
    ],
    out_specs=pl.BlockSpec(memory_space=pl.ANY),
    scratch_shapes=[pltpu.SemaphoreType.DMA] * _NQ,
)


def kernel(x, in_embed):
    return _tc_gather(x.astype(jnp.int32), in_embed)
